# baseline (speedup 1.0000x reference)
import functools

import jax
import jax.numpy as jnp
from jax import lax
from jax.experimental import pallas as pl
from jax.experimental.pallas import tpu as pltpu
from jax.experimental.pallas import tpu_sc as plsc

N_NODES = 10000
N_EDGES = 320000
D_FEAT = 128

NC = 2
NS = 16
NW = NC * NS
L = 16

D_AUG = D_FEAT + L
CHUNK = 128
CH_PER_W = 80
NCH_TOT = NW * CH_PER_W
E_PAD = NCH_TOT * CHUNK
N_PAD = 10240
ROWS_PER_TILE = N_PAD // NS


def _sc_scatter(x_aug, idx_pack):
    mesh = plsc.VectorSubcoreMesh(core_axis_name="c", subcore_axis_name="s")

    @functools.partial(
        pl.kernel,
        mesh=mesh,
        compiler_params=pltpu.CompilerParams(use_tc_tiling_on_sc=False),
        out_type=jax.ShapeDtypeStruct((NC * N_PAD, D_AUG), jnp.float32),
        scratch_types=[
            pltpu.VMEM_SHARED((N_PAD, D_AUG), jnp.float32),
            pltpu.VMEM((2, CHUNK), jnp.int32),
            pltpu.VMEM((2, CHUNK), jnp.int32),
            pltpu.VMEM((CHUNK, D_AUG), jnp.float32),
            pltpu.VMEM((CHUNK, D_AUG), jnp.float32),
            pltpu.VMEM((8, D_AUG), jnp.float32),
            pltpu.SemaphoreType.DMA,
            pltpu.SemaphoreType.DMA,
        ],
    )
    def k(x_hbm, pack_hbm, acc_out,
          acc_sp, idx0, idx1, msgs0, msgs1, zblk, sem0, sem1):
        cid = lax.axis_index("c")
        sid = lax.axis_index("s")
        wid = sid * NC + cid

        zero = jnp.zeros((L,), jnp.float32)
        for r in range(8):
            for cc in range(D_AUG // L):
                zblk[r, pl.ds(cc * L, L)] = zero

        tbase = sid * ROWS_PER_TILE

        def zero_body(j, _):
            pltpu.sync_copy(zblk, acc_sp.at[pl.ds(tbase + j * 8, 8)])
            return _

        lax.fori_loop(0, ROWS_PER_TILE // 8, zero_body, None)
        plsc.subcore_barrier()

        kbase = wid * CH_PER_W

        pltpu.sync_copy(pack_hbm.at[pl.ds(2 * kbase, 2)], idx0)
        pltpu.async_copy(x_hbm.at[idx0.at[0]], msgs0, sem0)

        def pipe_body(g, _):
            k1 = kbase + 2 * g + 1
            pltpu.sync_copy(pack_hbm.at[pl.ds(2 * k1, 2)], idx1)
            pltpu.async_copy(x_hbm.at[idx1.at[0]], msgs1, sem1)
            pltpu.make_async_copy(x_hbm.at[idx0.at[0]], msgs0, sem0).wait()
            pltpu.sync_copy(msgs0, acc_sp.at[idx0.at[1]], add=True)

            @pl.when(g + 1 < CH_PER_W // 2)
            def _prefetch():
                k2 = k1 + 1
                pltpu.sync_copy(pack_hbm.at[pl.ds(2 * k2, 2)], idx0)
                pltpu.async_copy(x_hbm.at[idx0.at[0]], msgs0, sem0)

            pltpu.make_async_copy(x_hbm.at[idx1.at[0]], msgs1, sem1).wait()
            pltpu.sync_copy(msgs1, acc_sp.at[idx1.at[1]], add=True)
            return _

        lax.fori_loop(0, CH_PER_W // 2, pipe_body, None)
        plsc.subcore_barrier()

        obase = cid * N_PAD + tbase
        pltpu.sync_copy(acc_sp.at[pl.ds(tbase, ROWS_PER_TILE)],
                        acc_out.at[pl.ds(obase, ROWS_PER_TILE)])

    return k(x_aug, idx_pack)


def _combine_body(acc_ref, out_ref):
    s = acc_ref[:N_NODES, :] + acc_ref[N_PAD:N_PAD + N_NODES, :]
    out_ref[...] = s[:, :D_FEAT] / jnp.maximum(s[:, D_FEAT:D_FEAT + 1], 1.0)


def kernel(x, edge_index):
    row = edge_index[0]
    col = edge_index[1]
    pad = E_PAD - N_EDGES
    row_pad = jnp.concatenate([row, jnp.zeros((pad,), jnp.int32)])
    dummy = N_NODES + (jnp.arange(pad, dtype=jnp.int32) % (N_PAD - N_NODES))
    col_pad = jnp.concatenate([col, dummy])
    idx_pack = jnp.stack(
        [row_pad.reshape(NCH_TOT, CHUNK), col_pad.reshape(NCH_TOT, CHUNK)],
        axis=1).reshape(2 * NCH_TOT, CHUNK)
    ones_col = jnp.zeros((N_NODES, L), jnp.float32).at[:, 0].set(1.0)
    x_aug = jnp.concatenate([x, ones_col], axis=1)

    acc = _sc_scatter(x_aug, idx_pack)

    out = pl.pallas_call(
        _combine_body,
        out_shape=jax.ShapeDtypeStruct((N_NODES, D_FEAT), jnp.float32),
    )(acc)
    return out


# speedup vs baseline: 1.0791x; 1.0791x over previous
import functools

import jax
import jax.numpy as jnp
from jax import lax
from jax.experimental import pallas as pl
from jax.experimental.pallas import tpu as pltpu
from jax.experimental.pallas import tpu_sc as plsc

N_NODES = 10000
N_EDGES = 320000
D_FEAT = 128

NC = 2
NS = 16
NW = NC * NS
L = 16

D_AUG = D_FEAT + L
CHUNK = 128
CH_W0 = 124
CH_W1 = 36
NCH_TOT = NS * (CH_W0 + CH_W1)
E_PAD = NCH_TOT * CHUNK
N_PAD = 10240
ROWS_PER_TILE = N_PAD // NS


def _sc_scatter(x_aug, idx_pack):
    mesh = plsc.VectorSubcoreMesh(core_axis_name="c", subcore_axis_name="s")

    @functools.partial(
        pl.kernel,
        mesh=mesh,
        compiler_params=pltpu.CompilerParams(use_tc_tiling_on_sc=False),
        out_type=jax.ShapeDtypeStruct((NC * N_PAD, D_AUG), jnp.float32),
        scratch_types=[
            pltpu.VMEM_SHARED((N_PAD, D_AUG), jnp.float32),
            pltpu.VMEM((2, CHUNK), jnp.int32),
            pltpu.VMEM((2, CHUNK), jnp.int32),
            pltpu.VMEM((CHUNK, D_AUG), jnp.float32),
            pltpu.VMEM((CHUNK, D_AUG), jnp.float32),
            pltpu.VMEM((8, D_AUG), jnp.float32),
            pltpu.SemaphoreType.DMA,
            pltpu.SemaphoreType.DMA,
        ],
    )
    def k(x_hbm, pack_hbm, acc_out,
          acc_sp, idx0, idx1, msgs0, msgs1, zblk, sem0, sem1):
        cid = lax.axis_index("c")
        sid = lax.axis_index("s")

        zero = jnp.zeros((L,), jnp.float32)
        for r in range(8):
            for cc in range(D_AUG // L):
                zblk[r, pl.ds(cc * L, L)] = zero

        tbase = sid * ROWS_PER_TILE

        def zero_body(j, _):
            pltpu.sync_copy(zblk, acc_sp.at[pl.ds(tbase + j * 8, 8)])
            return _

        lax.fori_loop(0, ROWS_PER_TILE // 8, zero_body, None)
        plsc.subcore_barrier()

        kbase = jnp.where(cid == 0, sid * CH_W0, NS * CH_W0 + sid * CH_W1)
        half = jnp.where(cid == 0, CH_W0 // 2, CH_W1 // 2)

        pltpu.sync_copy(pack_hbm.at[pl.ds(2 * kbase, 2)], idx0)
        pltpu.async_copy(x_hbm.at[idx0.at[0]], msgs0, sem0)

        def pipe_body(g, _):
            k1 = kbase + 2 * g + 1
            pltpu.sync_copy(pack_hbm.at[pl.ds(2 * k1, 2)], idx1)
            pltpu.async_copy(x_hbm.at[idx1.at[0]], msgs1, sem1)
            pltpu.make_async_copy(x_hbm.at[idx0.at[0]], msgs0, sem0).wait()
            pltpu.sync_copy(msgs0, acc_sp.at[idx0.at[1]], add=True)

            @pl.when(g + 1 < half)
            def _prefetch():
                k2 = k1 + 1
                pltpu.sync_copy(pack_hbm.at[pl.ds(2 * k2, 2)], idx0)
                pltpu.async_copy(x_hbm.at[idx0.at[0]], msgs0, sem0)

            pltpu.make_async_copy(x_hbm.at[idx1.at[0]], msgs1, sem1).wait()
            pltpu.sync_copy(msgs1, acc_sp.at[idx1.at[1]], add=True)
            return _

        lax.fori_loop(0, half, pipe_body, None)
        plsc.subcore_barrier()

        obase = cid * N_PAD + tbase
        pltpu.sync_copy(acc_sp.at[pl.ds(tbase, ROWS_PER_TILE)],
                        acc_out.at[pl.ds(obase, ROWS_PER_TILE)])

    return k(x_aug, idx_pack)


def _combine_body(acc_ref, out_ref):
    s = acc_ref[:N_NODES, :] + acc_ref[N_PAD:N_PAD + N_NODES, :]
    out_ref[...] = s[:, :D_FEAT] / jnp.maximum(s[:, D_FEAT:D_FEAT + 1], 1.0)


def kernel(x, edge_index):
    row = edge_index[0]
    col = edge_index[1]
    pad = E_PAD - N_EDGES
    row_pad = jnp.concatenate([row, jnp.zeros((pad,), jnp.int32)])
    dummy = N_NODES + (jnp.arange(pad, dtype=jnp.int32) % (N_PAD - N_NODES))
    col_pad = jnp.concatenate([col, dummy])
    idx_pack = jnp.stack(
        [row_pad.reshape(NCH_TOT, CHUNK), col_pad.reshape(NCH_TOT, CHUNK)],
        axis=1).reshape(2 * NCH_TOT, CHUNK)
    ones_col = jnp.zeros((N_NODES, L), jnp.float32).at[:, 0].set(1.0)
    x_aug = jnp.concatenate([x, ones_col], axis=1)

    acc = _sc_scatter(x_aug, idx_pack)

    out = pl.pallas_call(
        _combine_body,
        out_shape=jax.ShapeDtypeStruct((N_NODES, D_FEAT), jnp.float32),
    )(acc)
    return out
